# baseline (speedup 1.0000x reference)
import functools

import jax
import jax.numpy as jnp
from jax import lax
from jax.experimental import pallas as pl
from jax.experimental.pallas import tpu as pltpu
from jax.experimental.pallas import tpu_sc as plsc

N = 10000
E = 320000
D_IN = 128
D_HID = 64
D_OUT = 128

NC = 2
NS = 16
NW = NC * NS
CB = 128
K = 80
EPT = K * CB
EP = NW * EPT
NP = 10240
RPT = NP // NS
NZC = RPT // CB

_f32 = jnp.float32


def _sc_mesh():
    return plsc.VectorSubcoreMesh(core_axis_name="c", subcore_axis_name="s")


def _deg_body(src3, dst3, out_o, out_i, sidx, didx, vals, acc_o, acc_i,
              so, si):
    cid = lax.axis_index("c")
    sid = lax.axis_index("s")
    wid = cid * NS + sid

    z16 = jnp.zeros((16,), _f32)
    o16 = jnp.ones((16,), _f32)

    def zfill(i, _):
        vals[i, :] = z16
        return 0

    lax.fori_loop(0, CB, zfill, 0)
    for k in range(NZC):
        r0 = sid * RPT + k * CB
        pltpu.sync_copy(vals, acc_o.at[pl.ds(r0, CB)])
        pltpu.sync_copy(vals, acc_i.at[pl.ds(r0, CB)])

    def ofill(i, _):
        vals[i, :] = o16
        return 0

    lax.fori_loop(0, CB, ofill, 0)
    pltpu.sync_copy(src3.at[wid], sidx)
    pltpu.sync_copy(dst3.at[wid], didx)
    plsc.subcore_barrier()

    G = 8

    def group(g, _):
        for i in range(G):
            pltpu.async_copy(vals, acc_o.at[sidx.at[g * G + i]], so, add=True)
            pltpu.async_copy(vals, acc_i.at[didx.at[g * G + i]], si, add=True)
        for i in range(G):
            pltpu.make_async_copy(vals, acc_o.at[sidx.at[0]], so).wait()
            pltpu.make_async_copy(vals, acc_i.at[didx.at[0]], si).wait()
        return 0

    lax.fori_loop(0, K // G, group, 0)
    plsc.subcore_barrier()
    for k in range(NZC):
        r0 = sid * RPT + k * CB
        pltpu.sync_copy(acc_o.at[pl.ds(r0, CB)], out_o.at[cid, pl.ds(r0, CB)])
        pltpu.sync_copy(acc_i.at[pl.ds(r0, CB)], out_i.at[cid, pl.ds(r0, CB)])


def _sc_degrees(src3, dst3):
    kfn = pl.kernel(
        _deg_body,
        out_type=(
            jax.ShapeDtypeStruct((NC, NP, 16), _f32),
            jax.ShapeDtypeStruct((NC, NP, 16), _f32),
        ),
        mesh=_sc_mesh(),
        compiler_params=pltpu.CompilerParams(use_tc_tiling_on_sc=False),
        scratch_types=[
            pltpu.VMEM((K, CB), jnp.int32),
            pltpu.VMEM((K, CB), jnp.int32),
            pltpu.VMEM((CB, 16), _f32),
            pltpu.VMEM_SHARED((NP, 16), _f32),
            pltpu.VMEM_SHARED((NP, 16), _f32),
            pltpu.SemaphoreType.DMA,
            pltpu.SemaphoreType.DMA,
        ],
    )
    return kfn(src3, dst3)


def _edge_body(table, src3, dst3, out, sidx, didx, rows0, rows1, acc,
               sem0, sem1):
    cid = lax.axis_index("c")
    sid = lax.axis_index("s")
    wid = cid * NS + sid

    z16 = jnp.zeros((16,), _f32)

    def zrow(i, _):
        for c in range(D_HID // 16):
            rows0[i, pl.ds(c * 16, 16)] = z16
        return 0

    lax.fori_loop(0, CB, zrow, 0)
    for k in range(NZC):
        pltpu.sync_copy(rows0, acc.at[pl.ds(sid * RPT + k * CB, CB)])
    pltpu.sync_copy(src3.at[wid], sidx)
    pltpu.sync_copy(dst3.at[wid], didx)
    plsc.subcore_barrier()

    pltpu.async_copy(table.at[sidx.at[0]], rows0, sem0)
    pltpu.async_copy(table.at[sidx.at[1]], rows1, sem1)

    def step(g, _):
        j0 = 2 * g
        pltpu.make_async_copy(table.at[sidx.at[j0]], rows0, sem0).wait()
        pltpu.sync_copy(rows0, acc.at[didx.at[j0]], add=True)
        pltpu.async_copy(table.at[sidx.at[j0 + 2]], rows0, sem0)
        pltpu.make_async_copy(table.at[sidx.at[j0 + 1]], rows1, sem1).wait()
        pltpu.sync_copy(rows1, acc.at[didx.at[j0 + 1]], add=True)
        pltpu.async_copy(table.at[sidx.at[j0 + 3]], rows1, sem1)
        return 0

    lax.fori_loop(0, (K - 2) // 2, step, 0)
    pltpu.make_async_copy(table.at[sidx.at[K - 2]], rows0, sem0).wait()
    pltpu.sync_copy(rows0, acc.at[didx.at[K - 2]], add=True)
    pltpu.make_async_copy(table.at[sidx.at[K - 1]], rows1, sem1).wait()
    pltpu.sync_copy(rows1, acc.at[didx.at[K - 1]], add=True)
    plsc.subcore_barrier()
    for k in range(NZC):
        r0 = sid * RPT + k * CB
        pltpu.sync_copy(acc.at[pl.ds(r0, CB)], out.at[cid, pl.ds(r0, CB)])


def _sc_edge(table, src3, dst3):
    kfn = pl.kernel(
        _edge_body,
        out_type=jax.ShapeDtypeStruct((NC, NP, D_HID), _f32),
        mesh=_sc_mesh(),
        compiler_params=pltpu.CompilerParams(use_tc_tiling_on_sc=False),
        scratch_types=[
            pltpu.VMEM((K, CB), jnp.int32),
            pltpu.VMEM((K, CB), jnp.int32),
            pltpu.VMEM((CB, D_HID), _f32),
            pltpu.VMEM((CB, D_HID), _f32),
            pltpu.VMEM_SHARED((NP, D_HID), _f32),
            pltpu.SemaphoreType.DMA,
            pltpu.SemaphoreType.DMA,
        ],
    )
    return kfn(table, src3, dst3)


def _tc_premix(featsp, W0, do0, do1):
    def body(f, w, d0, d1, o):
        deg = d0[...][:, 0] + d1[...][:, 0]
        s = lax.rsqrt(jnp.maximum(deg, 1.0))
        o[...] = jnp.dot(f[...] * s[:, None], w[...],
                         preferred_element_type=_f32)

    return pl.pallas_call(
        body, out_shape=jax.ShapeDtypeStruct((NP, D_HID), _f32)
    )(featsp, W0, do0, do1)


def _tc_mid(a0, a1, di0, di1, do0, do1, b0):
    def body(a0r, a1r, di0r, di1r, do0r, do1r, br, h_ref, t_ref):
        s_in = lax.rsqrt(jnp.maximum(di0r[...][:, 0] + di1r[...][:, 0], 1.0))
        s_out = lax.rsqrt(jnp.maximum(do0r[...][:, 0] + do1r[...][:, 0], 1.0))
        agg = a0r[...] + a1r[...]
        h = jnp.maximum(agg * s_in[:, None] + br[...], 0.0)
        h_ref[...] = h
        row = lax.broadcasted_iota(jnp.int32, (NP, 1), 0)
        t_ref[...] = jnp.where(row < N, h * s_out[:, None], 0.0)

    return pl.pallas_call(
        body,
        out_shape=(
            jax.ShapeDtypeStruct((NP, D_HID), _f32),
            jax.ShapeDtypeStruct((NP, D_HID), _f32),
        ),
    )(a0, a1, di0, di1, do0, do1, b0)


def _tc_out(a0, a1, di0, di1, W1, b1):
    def body(a0r, a1r, di0r, di1r, wr, br, o_ref):
        s_in = lax.rsqrt(jnp.maximum(di0r[...][:, 0] + di1r[...][:, 0], 1.0))
        agg = (a0r[...] + a1r[...]) * s_in[:, None]
        o_ref[...] = jnp.dot(agg, wr[...], preferred_element_type=_f32) + br[...]

    return pl.pallas_call(
        body, out_shape=jax.ShapeDtypeStruct((NP, D_OUT), _f32)
    )(a0, a1, di0, di1, W1, b1)


def kernel(feats, edge_index, W0, b0, W1, b1):
    src = edge_index[0]
    dst = edge_index[1]
    pad = jnp.full((EP - E,), N, jnp.int32)
    src3 = jnp.concatenate([src, pad]).reshape(NW, K, CB)
    dst3 = jnp.concatenate([dst, pad]).reshape(NW, K, CB)
    featsp = jnp.pad(feats, ((0, NP - N), (0, 0)))

    dpo, dpi = _sc_degrees(src3, dst3)
    do0, do1 = dpo[0], dpo[1]
    di0, di1 = dpi[0], dpi[1]

    x0 = _tc_premix(featsp, W0, do0, do1)
    a0 = _sc_edge(x0, src3, dst3)
    hemb, t = _tc_mid(a0[0], a0[1], di0, di1, do0, do1,
                      b0.reshape(1, D_HID))
    a1 = _sc_edge(t, src3, dst3)
    out = _tc_out(a1[0], a1[1], di0, di1, W1, b1.reshape(1, D_OUT))
    return (hemb[:N], out[:N])


# speedup vs baseline: 1.0437x; 1.0437x over previous
import functools

import jax
import jax.numpy as jnp
from jax import lax
from jax.experimental import pallas as pl
from jax.experimental.pallas import tpu as pltpu
from jax.experimental.pallas import tpu_sc as plsc

N = 10000
E = 320000
D_IN = 128
D_HID = 64
D_OUT = 128

NC = 2
NS = 16
NW = NC * NS
CB = 128
NCHUNK = 2560
EP = NCHUNK * CB
KE0, KE1 = 118, 42
KD0, KD1 = 96, 64
NP = 10240
RPT = NP // NS
NZC = RPT // CB

_f32 = jnp.float32


def _sc_mesh():
    return plsc.VectorSubcoreMesh(core_axis_name="c", subcore_axis_name="s")


def _deg_body(src2, dst2, out_o, out_i, sidx, didx, vals, acc_o, acc_i,
              so, si):
    cid = lax.axis_index("c")
    sid = lax.axis_index("s")

    z16 = jnp.zeros((16,), _f32)
    o16 = jnp.ones((16,), _f32)

    def zfill(i, _):
        vals[i, :] = z16
        return 0

    lax.fori_loop(0, CB, zfill, 0)
    for k in range(NZC):
        r0 = sid * RPT + k * CB
        pltpu.sync_copy(vals, acc_o.at[pl.ds(r0, CB)])
        pltpu.sync_copy(vals, acc_i.at[pl.ds(r0, CB)])

    def ofill(i, _):
        vals[i, :] = o16
        return 0

    lax.fori_loop(0, CB, ofill, 0)
    plsc.subcore_barrier()

    G = 8

    def run(base, kc):
        pltpu.sync_copy(src2.at[pl.ds(base, kc)], sidx.at[pl.ds(0, kc)])
        pltpu.sync_copy(dst2.at[pl.ds(base, kc)], didx.at[pl.ds(0, kc)])

        def group(g, _):
            for i in range(G):
                pltpu.async_copy(vals, acc_o.at[sidx.at[g * G + i]], so,
                                 add=True)
                pltpu.async_copy(vals, acc_i.at[didx.at[g * G + i]], si,
                                 add=True)
            for i in range(G):
                pltpu.make_async_copy(vals, acc_o.at[sidx.at[0]], so).wait()
                pltpu.make_async_copy(vals, acc_i.at[didx.at[0]], si).wait()
            return 0

        lax.fori_loop(0, kc // G, group, 0)

    @pl.when(cid == 0)
    def _():
        run(sid * KD0, KD0)

    @pl.when(cid == 1)
    def _():
        run(NS * KD0 + sid * KD1, KD1)

    plsc.subcore_barrier()
    for k in range(NZC):
        r0 = sid * RPT + k * CB
        pltpu.sync_copy(acc_o.at[pl.ds(r0, CB)], out_o.at[cid, pl.ds(r0, CB)])
        pltpu.sync_copy(acc_i.at[pl.ds(r0, CB)], out_i.at[cid, pl.ds(r0, CB)])


def _sc_degrees(src2, dst2):
    kfn = pl.kernel(
        _deg_body,
        out_type=(
            jax.ShapeDtypeStruct((NC, NP, 16), _f32),
            jax.ShapeDtypeStruct((NC, NP, 16), _f32),
        ),
        mesh=_sc_mesh(),
        compiler_params=pltpu.CompilerParams(use_tc_tiling_on_sc=False),
        scratch_types=[
            pltpu.VMEM((KD0, CB), jnp.int32),
            pltpu.VMEM((KD0, CB), jnp.int32),
            pltpu.VMEM((CB, 16), _f32),
            pltpu.VMEM_SHARED((NP, 16), _f32),
            pltpu.VMEM_SHARED((NP, 16), _f32),
            pltpu.SemaphoreType.DMA,
            pltpu.SemaphoreType.DMA,
        ],
    )
    return kfn(src2, dst2)


def _edge_body(table, src2, dst2, out, sidx, didx, rows0, rows1, acc,
               sem0, sem1):
    cid = lax.axis_index("c")
    sid = lax.axis_index("s")

    z16 = jnp.zeros((16,), _f32)

    def zrow(i, _):
        for c in range(D_HID // 16):
            rows0[i, pl.ds(c * 16, 16)] = z16
        return 0

    lax.fori_loop(0, CB, zrow, 0)
    for k in range(NZC):
        pltpu.sync_copy(rows0, acc.at[pl.ds(sid * RPT + k * CB, CB)])
    plsc.subcore_barrier()

    def run(base, kc):
        pltpu.sync_copy(src2.at[pl.ds(base, kc)], sidx.at[pl.ds(0, kc)])
        pltpu.sync_copy(dst2.at[pl.ds(base, kc)], didx.at[pl.ds(0, kc)])
        pltpu.async_copy(table.at[sidx.at[0]], rows0, sem0)
        pltpu.async_copy(table.at[sidx.at[1]], rows1, sem1)

        def step(g, _):
            j0 = 2 * g
            pltpu.make_async_copy(table.at[sidx.at[j0]], rows0, sem0).wait()
            pltpu.sync_copy(rows0, acc.at[didx.at[j0]], add=True)
            pltpu.async_copy(table.at[sidx.at[j0 + 2]], rows0, sem0)
            pltpu.make_async_copy(table.at[sidx.at[j0 + 1]], rows1,
                                  sem1).wait()
            pltpu.sync_copy(rows1, acc.at[didx.at[j0 + 1]], add=True)
            pltpu.async_copy(table.at[sidx.at[j0 + 3]], rows1, sem1)
            return 0

        lax.fori_loop(0, (kc - 2) // 2, step, 0)
        pltpu.make_async_copy(table.at[sidx.at[kc - 2]], rows0, sem0).wait()
        pltpu.sync_copy(rows0, acc.at[didx.at[kc - 2]], add=True)
        pltpu.make_async_copy(table.at[sidx.at[kc - 1]], rows1, sem1).wait()
        pltpu.sync_copy(rows1, acc.at[didx.at[kc - 1]], add=True)

    @pl.when(cid == 0)
    def _():
        run(sid * KE0, KE0)

    @pl.when(cid == 1)
    def _():
        run(NS * KE0 + sid * KE1, KE1)

    plsc.subcore_barrier()
    for k in range(NZC):
        r0 = sid * RPT + k * CB
        pltpu.sync_copy(acc.at[pl.ds(r0, CB)], out.at[cid, pl.ds(r0, CB)])


def _sc_edge(table, src2, dst2):
    kfn = pl.kernel(
        _edge_body,
        out_type=jax.ShapeDtypeStruct((NC, NP, D_HID), _f32),
        mesh=_sc_mesh(),
        compiler_params=pltpu.CompilerParams(use_tc_tiling_on_sc=False),
        scratch_types=[
            pltpu.VMEM((KE0, CB), jnp.int32),
            pltpu.VMEM((KE0, CB), jnp.int32),
            pltpu.VMEM((CB, D_HID), _f32),
            pltpu.VMEM((CB, D_HID), _f32),
            pltpu.VMEM_SHARED((NP, D_HID), _f32),
            pltpu.SemaphoreType.DMA,
            pltpu.SemaphoreType.DMA,
        ],
    )
    return kfn(table, src2, dst2)


def _tc_premix(feats, W0, do0, do1):
    def body(f, w, d0, d1, o):
        deg = d0[...][:N, 0] + d1[...][:N, 0]
        s = lax.rsqrt(jnp.maximum(deg, 1.0))
        o[0:N, :] = jnp.dot(f[...] * s[:, None], w[...],
                            preferred_element_type=_f32)
        o[N:NP, :] = jnp.zeros((NP - N, D_HID), _f32)

    return pl.pallas_call(
        body, out_shape=jax.ShapeDtypeStruct((NP, D_HID), _f32)
    )(feats, W0, do0, do1)


def _tc_mid(a0, a1, di0, di1, do0, do1, b0):
    def body(a0r, a1r, di0r, di1r, do0r, do1r, br, h_ref, t_ref):
        s_in = lax.rsqrt(jnp.maximum(di0r[...][:, 0] + di1r[...][:, 0], 1.0))
        s_out = lax.rsqrt(jnp.maximum(do0r[...][:, 0] + do1r[...][:, 0], 1.0))
        agg = a0r[...] + a1r[...]
        h = jnp.maximum(agg * s_in[:, None] + br[...], 0.0)
        h_ref[...] = h
        row = lax.broadcasted_iota(jnp.int32, (NP, 1), 0)
        t_ref[...] = jnp.where(row < N, h * s_out[:, None], 0.0)

    return pl.pallas_call(
        body,
        out_shape=(
            jax.ShapeDtypeStruct((NP, D_HID), _f32),
            jax.ShapeDtypeStruct((NP, D_HID), _f32),
        ),
    )(a0, a1, di0, di1, do0, do1, b0)


def _tc_out(a0, a1, di0, di1, W1, b1):
    def body(a0r, a1r, di0r, di1r, wr, br, o_ref):
        s_in = lax.rsqrt(jnp.maximum(di0r[...][:, 0] + di1r[...][:, 0], 1.0))
        agg = (a0r[...] + a1r[...]) * s_in[:, None]
        o_ref[...] = jnp.dot(agg, wr[...], preferred_element_type=_f32) + br[...]

    return pl.pallas_call(
        body, out_shape=jax.ShapeDtypeStruct((NP, D_OUT), _f32)
    )(a0, a1, di0, di1, W1, b1)


def kernel(feats, edge_index, W0, b0, W1, b1):
    src = edge_index[0]
    dst = edge_index[1]
    pad = jnp.full((EP - E,), N, jnp.int32)
    src2 = jnp.concatenate([src, pad]).reshape(NCHUNK, CB)
    dst2 = jnp.concatenate([dst, pad]).reshape(NCHUNK, CB)

    dpo, dpi = _sc_degrees(src2, dst2)
    do0, do1 = dpo[0], dpo[1]
    di0, di1 = dpi[0], dpi[1]

    x0 = _tc_premix(feats, W0, do0, do1)
    a0 = _sc_edge(x0, src2, dst2)
    hemb, t = _tc_mid(a0[0], a0[1], di0, di1, do0, do1,
                      b0.reshape(1, D_HID))
    a1 = _sc_edge(t, src2, dst2)
    out = _tc_out(a1[0], a1[1], di0, di1, W1, b1.reshape(1, D_OUT))
    return (hemb[:N], out[:N])
